# baseline (speedup 1.0000x reference)
import functools

import jax
import jax.numpy as jnp
from jax import lax
from jax.experimental import pallas as pl
from jax.experimental.pallas import tpu as pltpu
from jax.experimental.pallas import tpu_sc as plsc

VOCAB = 1000000
EMBED = 64
B = 4096
L = 200
NCLS = 5

NUM_CORES = 2
NUM_SUBCORES = 16
NW = NUM_CORES * NUM_SUBCORES
BPW = B // NW
WAVE = 8
TCOLS = 16384
HALF = -(-(VOCAB // 2) // TCOLS) * TCOLS


def _transpose_body(a_ref, b_ref, o_ref):
    ident = jnp.eye(EMBED, dtype=jnp.float32)
    dims = (((0,), (0,)), ((), ()))
    o_ref[:, :EMBED] = lax.dot_general(
        a_ref[...], ident, dims, precision=lax.Precision.DEFAULT,
        preferred_element_type=jnp.float32)
    o_ref[:, EMBED:] = lax.dot_general(
        b_ref[...], ident, dims, precision=lax.Precision.DEFAULT,
        preferred_element_type=jnp.float32)


def _to_row_major(table):
    paired = pl.pallas_call(
        _transpose_body,
        grid=(HALF // TCOLS,),
        in_specs=[pl.BlockSpec((EMBED, TCOLS), lambda i: (0, i)),
                  pl.BlockSpec((EMBED, TCOLS),
                               lambda i: (0, jnp.minimum(
                                   i + HALF // TCOLS,
                                   pl.cdiv(VOCAB, TCOLS) - 1)))],
        out_specs=pl.BlockSpec((TCOLS, 2 * EMBED), lambda i: (i, 0)),
        out_shape=jax.ShapeDtypeStruct((HALF, 2 * EMBED), jnp.float32),
    )(table.T, table.T)
    return paired.reshape(2 * HALF, EMBED)


def _pool_sums(idx_t, table, t):
    mesh = plsc.VectorSubcoreMesh(
        core_axis_name="c", subcore_axis_name="s",
        num_cores=NUM_CORES, num_subcores=NUM_SUBCORES)

    @functools.partial(
        pl.kernel,
        out_type=jax.ShapeDtypeStruct((B, EMBED), jnp.float32),
        mesh=mesh,
        scratch_types=[
            pltpu.VMEM((L, BPW), jnp.int32),
            pltpu.VMEM((BPW, EMBED), jnp.float32),
            pltpu.SemaphoreType.DMA,
        ],
        compiler_params=pltpu.CompilerParams(use_tc_tiling_on_sc=False),
    )
    def pool(idx_hbm, tab_hbm, out_hbm, idx_v, acc_v, sem):
        wid = lax.axis_index("c") * NUM_SUBCORES + lax.axis_index("s")
        base = wid * BPW
        zeros = jnp.zeros((16,), jnp.float32)
        pltpu.sync_copy(idx_hbm.at[t, :, pl.ds(base, BPW)], idx_v)

        @pl.loop(0, BPW)
        def _zero(i):
            for j in range(EMBED // 16):
                acc_v[i, pl.ds(j * 16, 16)] = zeros

        @pl.loop(0, L, step=WAVE)
        def _wave(l0):
            cps = [
                pltpu.async_copy(tab_hbm.at[idx_v.at[l0 + j]], acc_v, sem,
                                 add=True)
                for j in range(WAVE)
            ]
            for cp in cps:
                cp.wait()

        pltpu.sync_copy(acc_v, out_hbm.at[pl.ds(base, BPW), :])

    return pool(idx_t, table)


def _mlp_body(x0_ref, x1_ref, w1_ref, b1_ref, w2_ref, b2_ref, o_ref):
    inv_l = 1.0 / L
    x0 = x0_ref[...] * inv_l
    x1 = x1_ref[...] * inv_l
    w1 = w1_ref[...]
    h = (lax.dot_general(x0, w1[:, :EMBED], (((1,), (1,)), ((), ())),
                         preferred_element_type=jnp.float32)
         + lax.dot_general(x1, w1[:, EMBED:], (((1,), (1,)), ((), ())),
                           preferred_element_type=jnp.float32))
    h = jnp.maximum(h + b1_ref[...], 0.0)
    logits = lax.dot_general(h, w2_ref[...], (((1,), (1,)), ((), ())),
                             preferred_element_type=jnp.float32)
    logits = logits + b2_ref[...]
    m = jnp.max(logits, axis=1, keepdims=True)
    e = jnp.exp(logits - m)
    o_ref[...] = e / jnp.sum(e, axis=1, keepdims=True)


def kernel(inputs, embed_bow, embed_bigram, W1, b1, W2, b2):
    ids = inputs.astype(jnp.int32)
    phys = jnp.where(ids < HALF, 2 * ids, 2 * (ids - HALF) + 1)
    idx_t = jnp.transpose(phys, (0, 2, 1))
    bow_rm = _to_row_major(embed_bow)
    big_rm = _to_row_major(embed_bigram)
    feat0 = _pool_sums(idx_t, bow_rm, 0)
    feat1 = _pool_sums(idx_t, big_rm, 1)
    out = pl.pallas_call(
        _mlp_body,
        out_shape=jax.ShapeDtypeStruct((B, NCLS), jnp.float32),
    )(feat0, feat1, W1, b1.reshape(1, EMBED), W2, b2.reshape(1, NCLS))
    return out


# speedup vs baseline: 1.0015x; 1.0015x over previous
import functools

import jax
import jax.numpy as jnp
from jax import lax
from jax.experimental import pallas as pl
from jax.experimental.pallas import tpu as pltpu
from jax.experimental.pallas import tpu_sc as plsc

VOCAB = 1000000
EMBED = 64
B = 4096
L = 200
NCLS = 5

NUM_CORES = 2
NUM_SUBCORES = 16
NW = NUM_CORES * NUM_SUBCORES
BPW = B // NW
WAVE = 8
TCOLS = 16384
HALF = -(-(VOCAB // 2) // TCOLS) * TCOLS


def _transpose_body(a_ref, b_ref, o_ref):
    o_ref[:, :EMBED] = a_ref[...].T
    o_ref[:, EMBED:] = b_ref[...].T


def _to_row_major(table):
    paired = pl.pallas_call(
        _transpose_body,
        grid=(HALF // TCOLS,),
        in_specs=[pl.BlockSpec((EMBED, TCOLS), lambda i: (0, i)),
                  pl.BlockSpec((EMBED, TCOLS),
                               lambda i: (0, jnp.minimum(
                                   i + HALF // TCOLS,
                                   pl.cdiv(VOCAB, TCOLS) - 1)))],
        out_specs=pl.BlockSpec((TCOLS, 2 * EMBED), lambda i: (i, 0)),
        out_shape=jax.ShapeDtypeStruct((HALF, 2 * EMBED), jnp.float32),
    )(table.T, table.T)
    return paired.reshape(2 * HALF, EMBED)


def _pool_sums(idx_t, table, t):
    mesh = plsc.VectorSubcoreMesh(
        core_axis_name="c", subcore_axis_name="s",
        num_cores=NUM_CORES, num_subcores=NUM_SUBCORES)

    @functools.partial(
        pl.kernel,
        out_type=jax.ShapeDtypeStruct((B, EMBED), jnp.float32),
        mesh=mesh,
        scratch_types=[
            pltpu.VMEM((L, BPW), jnp.int32),
            pltpu.VMEM((BPW, EMBED), jnp.float32),
            pltpu.SemaphoreType.DMA,
        ],
        compiler_params=pltpu.CompilerParams(use_tc_tiling_on_sc=False),
    )
    def pool(idx_hbm, tab_hbm, out_hbm, idx_v, acc_v, sem):
        wid = lax.axis_index("c") * NUM_SUBCORES + lax.axis_index("s")
        base = wid * BPW
        zeros = jnp.zeros((16,), jnp.float32)
        pltpu.sync_copy(idx_hbm.at[t, :, pl.ds(base, BPW)], idx_v)

        @pl.loop(0, BPW)
        def _zero(i):
            for j in range(EMBED // 16):
                acc_v[i, pl.ds(j * 16, 16)] = zeros

        @pl.loop(0, L, step=WAVE)
        def _wave(l0):
            cps = [
                pltpu.async_copy(tab_hbm.at[idx_v.at[l0 + j]], acc_v, sem,
                                 add=True)
                for j in range(WAVE)
            ]
            for cp in cps:
                cp.wait()

        pltpu.sync_copy(acc_v, out_hbm.at[pl.ds(base, BPW), :])

    return pool(idx_t, table)


def _mlp_body(x0_ref, x1_ref, w1_ref, b1_ref, w2_ref, b2_ref, o_ref):
    inv_l = 1.0 / L
    x0 = x0_ref[...] * inv_l
    x1 = x1_ref[...] * inv_l
    w1 = w1_ref[...]
    h = (lax.dot_general(x0, w1[:, :EMBED], (((1,), (1,)), ((), ())),
                         preferred_element_type=jnp.float32)
         + lax.dot_general(x1, w1[:, EMBED:], (((1,), (1,)), ((), ())),
                           preferred_element_type=jnp.float32))
    h = jnp.maximum(h + b1_ref[...], 0.0)
    logits = lax.dot_general(h, w2_ref[...], (((1,), (1,)), ((), ())),
                             preferred_element_type=jnp.float32)
    logits = logits + b2_ref[...]
    m = jnp.max(logits, axis=1, keepdims=True)
    e = jnp.exp(logits - m)
    o_ref[...] = e / jnp.sum(e, axis=1, keepdims=True)


def kernel(inputs, embed_bow, embed_bigram, W1, b1, W2, b2):
    ids = inputs.astype(jnp.int32)
    phys = jnp.where(ids < HALF, 2 * ids, 2 * (ids - HALF) + 1)
    idx_t = jnp.transpose(phys, (0, 2, 1))
    bow_rm = _to_row_major(embed_bow)
    big_rm = _to_row_major(embed_bigram)
    feat0 = _pool_sums(idx_t, bow_rm, 0)
    feat1 = _pool_sums(idx_t, big_rm, 1)
    out = pl.pallas_call(
        _mlp_body,
        out_shape=jax.ShapeDtypeStruct((B, NCLS), jnp.float32),
    )(feat0, feat1, W1, b1.reshape(1, EMBED), W2, b2.reshape(1, NCLS))
    return out
